# baseline (speedup 1.0000x reference)
import functools

import jax
import jax.numpy as jnp
from jax import lax
from jax.experimental import pallas as pl
from jax.experimental.pallas import tpu as pltpu
from jax.experimental.pallas import tpu_sc as plsc

_IN_DIM = 100000
_OUT_DIM = 64
_PAD_DIM = 128

_NC = 2
_NS = 16
_NW = _NC * _NS
_LANES = 16
_SUB = 8

_BATCH = 4096
_SEQ = 200
_B = _BATCH * _SEQ
_BPW = _B // _NW
_CH = 40
_NCH = _BPW // _CH
_CLIP_CH = 128


def _sc_lookup(table_hbm, idx_hbm, out_hbm, idx_v, bufa0, bufa1,
               bufb0, bufb1, sem_i, sg0, sg1, ss0, ss1):
    wid = lax.axis_index("s") * _NC + lax.axis_index("c")
    base = wid * _BPW

    pltpu.async_copy(idx_hbm.at[wid], idx_v, sem_i).wait()

    def _adjust(j, carry):
        for k in range(_CLIP_CH // _LANES):
            off = j * _CLIP_CH + k * _LANES
            v = idx_v[pl.ds(off, _LANES)]
            v = jnp.minimum(jnp.maximum(v, -_IN_DIM), _IN_DIM) + _IN_DIM
            idx_v[pl.ds(off, _LANES)] = v
        return carry

    lax.fori_loop(0, _BPW // _CLIP_CH, _adjust, 0)

    bufa = (bufa0, bufa1)
    bufb = (bufb0, bufb1)
    sgs = (sg0, sg1)
    sss = (ss0, ss1)

    def start_gather(c, p):
        pltpu.async_copy(
            table_hbm.at[idx_v.at[pl.ds(c * _CH, _CH)]], bufa[p], sgs[p])

    def wait_gather(p):
        pltpu.make_async_copy(
            table_hbm.at[pl.ds(0, _CH)], bufa[p], sgs[p]).wait()

    def compact(p):
        def rows(t, carry):
            for s in range(_SUB):
                for k in range(_OUT_DIM // _LANES):
                    bufb[p][t * _SUB + s, pl.ds(k * _LANES, _LANES)] = (
                        bufa[p][t * _SUB + s, pl.ds(k * _LANES, _LANES)])
            return carry

        lax.fori_loop(0, _CH // _SUB, rows, 0)

    def start_scatter(c, p):
        row = base + c * _CH
        b = lax.div(row, _SEQ)
        s0 = lax.rem(row, _SEQ)
        pltpu.async_copy(
            bufb[p], out_hbm.at[b, pl.ds(s0, _CH)], sss[p])

    def wait_scatter(p):
        pltpu.make_async_copy(
            bufb[p], out_hbm.at[0, pl.ds(0, _CH)], sss[p]).wait()

    def step(c, p, first, last):
        wait_gather(p)
        if not first:
            wait_scatter(p)
        if not last:
            start_gather(c + 1, 1 - p)
        compact(p)
        start_scatter(c, p)

    start_gather(0, 0)

    def body(gg, carry):
        c0 = 2 * gg

        @pl.when(gg == 0)
        def _():
            step(c0, 0, first=True, last=False)
            step(c0 + 1, 1, first=True, last=False)

        @pl.when(jnp.logical_and(gg > 0, gg < _NCH // 2 - 1))
        def _():
            step(c0, 0, first=False, last=False)
            step(c0 + 1, 1, first=False, last=False)

        @pl.when(gg == _NCH // 2 - 1)
        def _():
            step(c0, 0, first=False, last=False)
            step(c0 + 1, 1, first=False, last=True)

        return carry

    lax.fori_loop(0, _NCH // 2, body, 0)
    wait_scatter(0)
    wait_scatter(1)


def kernel(inputs, embeddings):
    idx = inputs.astype(jnp.int32).reshape(_NW, _BPW)
    table = jnp.pad(embeddings, ((0, 0), (0, _PAD_DIM - _OUT_DIM)))
    mesh = plsc.VectorSubcoreMesh(core_axis_name="c", subcore_axis_name="s")
    call = functools.partial(
        pl.kernel,
        mesh=mesh,
        out_type=jax.ShapeDtypeStruct((_BATCH, _SEQ, _OUT_DIM), jnp.float32),
        scratch_types=[
            pltpu.VMEM((_BPW,), jnp.int32),
            pltpu.VMEM((_CH, _PAD_DIM), jnp.float32),
            pltpu.VMEM((_CH, _PAD_DIM), jnp.float32),
            pltpu.VMEM((_CH, _OUT_DIM), jnp.float32),
            pltpu.VMEM((_CH, _OUT_DIM), jnp.float32),
            pltpu.SemaphoreType.DMA,
            pltpu.SemaphoreType.DMA,
            pltpu.SemaphoreType.DMA,
            pltpu.SemaphoreType.DMA,
            pltpu.SemaphoreType.DMA,
        ],
        compiler_params=pltpu.CompilerParams(use_tc_tiling_on_sc=True),
    )(_sc_lookup)
    return call(table, idx)


# speedup vs baseline: 1.4973x; 1.4973x over previous
import functools

import jax
import jax.numpy as jnp
from jax import lax
from jax.experimental import pallas as pl
from jax.experimental.pallas import tpu as pltpu
from jax.experimental.pallas import tpu_sc as plsc

_IN_DIM = 100000
_OUT_DIM = 64
_PAD_DIM = 128

_NC = 2
_NS = 16
_NW = _NC * _NS
_LANES = 16
_SUB = 8

_BATCH = 4096
_SEQ = 200
_B = _BATCH * _SEQ
_BPW = _B // _NW
_CH = _SEQ
_NCH = _BPW // _CH
_CLIP_CH = 128


def _sc_lookup(table_hbm, idx_hbm, out_hbm, idx_v, bufa0, bufa1,
               bufb0, bufb1, sem_i, sg0, sg1, ss0, ss1):
    wid = lax.axis_index("s") * _NC + lax.axis_index("c")
    base = wid * _BPW

    pltpu.async_copy(idx_hbm.at[wid], idx_v, sem_i).wait()

    def _adjust(j, carry):
        for k in range(_CLIP_CH // _LANES):
            off = j * _CLIP_CH + k * _LANES
            v = idx_v[pl.ds(off, _LANES)]
            v = jnp.minimum(jnp.maximum(v, -_IN_DIM), _IN_DIM) + _IN_DIM
            idx_v[pl.ds(off, _LANES)] = v
        return carry

    lax.fori_loop(0, _BPW // _CLIP_CH, _adjust, 0)

    bufa = (bufa0, bufa1)
    bufb = (bufb0, bufb1)
    sgs = (sg0, sg1)
    sss = (ss0, ss1)

    def start_gather(c, p):
        pltpu.async_copy(
            table_hbm.at[idx_v.at[pl.ds(c * _CH, 128)]],
            bufa[p].at[pl.ds(0, 128)], sgs[p])
        pltpu.async_copy(
            table_hbm.at[idx_v.at[pl.ds(c * _CH + 128, _CH - 128)]],
            bufa[p].at[pl.ds(128, _CH - 128)], sgs[p])

    def wait_gather(p):
        pltpu.make_async_copy(
            table_hbm.at[pl.ds(0, _CH)], bufa[p], sgs[p]).wait()

    def compact(p):
        def rows(t, carry):
            for s in range(_SUB):
                for k in range(_OUT_DIM // _LANES):
                    bufb[p][t * _SUB + s, pl.ds(k * _LANES, _LANES)] = (
                        bufa[p][t * _SUB + s, pl.ds(k * _LANES, _LANES)])
            return carry

        lax.fori_loop(0, _CH // _SUB, rows, 0)

    def start_scatter(c, p):
        b = wid * _NCH + c
        pltpu.async_copy(bufb[p], out_hbm.at[b], sss[p])

    def wait_scatter(p):
        pltpu.make_async_copy(bufb[p], out_hbm.at[0], sss[p]).wait()

    def step(c, p, first, last):
        wait_gather(p)
        if not first:
            wait_scatter(p)
        if not last:
            start_gather(c + 1, 1 - p)
        compact(p)
        start_scatter(c, p)

    start_gather(0, 0)

    def body(gg, carry):
        c0 = 2 * gg

        @pl.when(gg == 0)
        def _():
            step(c0, 0, first=True, last=False)
            step(c0 + 1, 1, first=True, last=False)

        @pl.when(jnp.logical_and(gg > 0, gg < _NCH // 2 - 1))
        def _():
            step(c0, 0, first=False, last=False)
            step(c0 + 1, 1, first=False, last=False)

        @pl.when(gg == _NCH // 2 - 1)
        def _():
            step(c0, 0, first=False, last=False)
            step(c0 + 1, 1, first=False, last=True)

        return carry

    lax.fori_loop(0, _NCH // 2, body, 0)
    wait_scatter(0)
    wait_scatter(1)


def kernel(inputs, embeddings):
    idx = inputs.astype(jnp.int32).reshape(_NW, _BPW)
    table = jnp.pad(embeddings, ((0, 0), (0, _PAD_DIM - _OUT_DIM)))
    mesh = plsc.VectorSubcoreMesh(core_axis_name="c", subcore_axis_name="s")
    call = functools.partial(
        pl.kernel,
        mesh=mesh,
        out_type=jax.ShapeDtypeStruct((_BATCH, _SEQ, _OUT_DIM), jnp.float32),
        scratch_types=[
            pltpu.VMEM((_BPW,), jnp.int32),
            pltpu.VMEM((_CH, _PAD_DIM), jnp.float32),
            pltpu.VMEM((_CH, _PAD_DIM), jnp.float32),
            pltpu.VMEM((_CH, _OUT_DIM), jnp.float32),
            pltpu.VMEM((_CH, _OUT_DIM), jnp.float32),
            pltpu.SemaphoreType.DMA,
            pltpu.SemaphoreType.DMA,
            pltpu.SemaphoreType.DMA,
            pltpu.SemaphoreType.DMA,
            pltpu.SemaphoreType.DMA,
        ],
        compiler_params=pltpu.CompilerParams(use_tc_tiling_on_sc=True),
    )(_sc_lookup)
    return call(table, idx)
